# baseline (speedup 1.0000x reference)
import functools

import jax
import jax.numpy as jnp
from jax import lax
from jax.experimental import pallas as pl
from jax.experimental.pallas import tpu as pltpu
from jax.experimental.pallas import tpu_sc as plsc

_CHUNK = 16384
_DEPTH = 4


def _make_sc_copy(n):
    info = plsc.get_sparse_core_info()
    nc, ns = info.num_cores, info.num_subcores
    nw = nc * ns
    seg = n // nw
    steps = seg // _CHUNK
    mesh = plsc.VectorSubcoreMesh(core_axis_name="c", subcore_axis_name="s")

    @functools.partial(
        pl.kernel,
        mesh=mesh,
        out_type=[
            jax.ShapeDtypeStruct((3, n), jnp.float32),
            jax.ShapeDtypeStruct((3, n), jnp.float32),
            jax.ShapeDtypeStruct((n,), jnp.float32),
        ],
        scratch_types=(
            [pltpu.VMEM((1, _CHUNK), jnp.float32)] * _DEPTH
            + [pltpu.VMEM((_CHUNK,), jnp.float32)]
            + [pltpu.SemaphoreType.DMA, pltpu.SemaphoreType.DMA]
        ),
    )
    def k(x_hbm, r_hbm, d_hbm, xo_hbm, ro_hbm, do_hbm, *rest):
        bufs = list(rest[:_DEPTH])
        buf1 = rest[_DEPTH]
        sem_rd, sem_wr = rest[_DEPTH + 1], rest[_DEPTH + 2]
        wid = lax.axis_index("s") * nc + lax.axis_index("c")
        base = wid * seg

        ops = []
        for src, dst in ((x_hbm, xo_hbm), (r_hbm, ro_hbm)):
            for row in range(3):
                for c in range(steps):
                    sl = (pl.ds(row, 1), pl.ds(base + c * _CHUNK, _CHUNK))
                    ops.append((src.at[sl], dst.at[sl]))

        inflight = [None] * _DEPTH
        for i, (src, dst) in enumerate(ops):
            slot = i % _DEPTH
            if inflight[slot] is not None:
                inflight[slot].wait()
            rd = pltpu.async_copy(src, bufs[slot], sem_rd)
            rd.wait()
            inflight[slot] = pltpu.async_copy(bufs[slot], dst, sem_wr)
        for wr in inflight:
            wr.wait()

        for c in range(steps):
            sl = pl.ds(base + c * _CHUNK, _CHUNK)
            pltpu.sync_copy(d_hbm.at[sl], buf1)
            pltpu.sync_copy(buf1, do_hbm.at[sl])

    return k


def kernel(sampled_point_xyz, sampled_point_ray_direction, sampled_point_distance):
    n = sampled_point_xyz.shape[0]
    xt = sampled_point_xyz.T
    rt = sampled_point_ray_direction.T
    pos_t, ray_t, dists = _make_sc_copy(n)(xt, rt, sampled_point_distance)
    return (pos_t.T, ray_t.T, dists)


# speedup vs baseline: 1.2561x; 1.2561x over previous
import functools

import jax
import jax.numpy as jnp
from jax import lax
from jax.experimental import pallas as pl
from jax.experimental.pallas import tpu as pltpu
from jax.experimental.pallas import tpu_sc as plsc

_CHUNK = 16384
_BUFS = 4
_RA = 2


def _ring_copy(ops, bufs, sem_rd, sem_wr):
    k = len(ops)
    nb = len(bufs)
    ra = min(_RA, nb - 1)
    rds = [None] * k
    wrs = [None] * k
    for b in range(min(ra, k)):
        rds[b] = pltpu.async_copy(ops[b][0], bufs[b % nb], sem_rd)
    for i in range(k):
        j = i + ra
        if j < k:
            w = j - nb
            if w >= 0:
                wrs[w].wait()
            rds[j] = pltpu.async_copy(ops[j][0], bufs[j % nb], sem_rd)
        rds[i].wait()
        wrs[i] = pltpu.async_copy(bufs[i % nb], ops[i][1], sem_wr)
    for i in range(max(0, k - nb), k):
        if wrs[i] is not None:
            wrs[i].wait()


def _make_sc_copy(n):
    info = plsc.get_sparse_core_info()
    nc, ns = info.num_cores, info.num_subcores
    nw = nc * ns
    seg = n // nw
    steps = seg // _CHUNK
    mesh = plsc.VectorSubcoreMesh(core_axis_name="c", subcore_axis_name="s")

    @functools.partial(
        pl.kernel,
        mesh=mesh,
        out_type=[
            jax.ShapeDtypeStruct((3, n), jnp.float32),
            jax.ShapeDtypeStruct((3, n), jnp.float32),
            jax.ShapeDtypeStruct((n,), jnp.float32),
        ],
        scratch_types=(
            [pltpu.VMEM((1, _CHUNK), jnp.float32)] * _BUFS
            + [pltpu.VMEM((_CHUNK,), jnp.float32)] * 2
            + [pltpu.SemaphoreType.DMA, pltpu.SemaphoreType.DMA]
        ),
    )
    def k(x_hbm, r_hbm, d_hbm, xo_hbm, ro_hbm, do_hbm, *rest):
        bufs2 = list(rest[:_BUFS])
        bufs1 = list(rest[_BUFS:_BUFS + 2])
        sem_rd, sem_wr = rest[_BUFS + 2], rest[_BUFS + 3]
        wid = lax.axis_index("s") * nc + lax.axis_index("c")
        base = wid * seg

        ops2 = []
        for src, dst in ((x_hbm, xo_hbm), (r_hbm, ro_hbm)):
            for row in range(3):
                for c in range(steps):
                    sl = (pl.ds(row, 1), pl.ds(base + c * _CHUNK, _CHUNK))
                    ops2.append((src.at[sl], dst.at[sl]))
        _ring_copy(ops2, bufs2, sem_rd, sem_wr)

        ops1 = []
        for c in range(steps):
            sl = pl.ds(base + c * _CHUNK, _CHUNK)
            ops1.append((d_hbm.at[sl], do_hbm.at[sl]))
        _ring_copy(ops1, bufs1, sem_rd, sem_wr)

    return k


def kernel(sampled_point_xyz, sampled_point_ray_direction, sampled_point_distance):
    n = sampled_point_xyz.shape[0]
    xt = sampled_point_xyz.T
    rt = sampled_point_ray_direction.T
    pos_t, ray_t, dists = _make_sc_copy(n)(xt, rt, sampled_point_distance)
    return (pos_t.T, ray_t.T, dists)
